# Initial kernel scaffold; baseline (speedup 1.0000x reference)
import jax
import jax.numpy as jnp
from jax.experimental import pallas as pl


def kernel(x, edge_index, y, train_mask, Wl1, bl1, Wr1, br1, Wl2, bl2, Wr2, br2):
    raise NotImplementedError("write your pallas kernel here")


# speedup vs baseline: 3.6873x; 3.6873x over previous
import functools

import jax
import jax.numpy as jnp
from jax import lax
from jax.experimental import pallas as pl
from jax.experimental.pallas import tpu as pltpu
from jax.experimental.pallas import tpu_sc as plsc

N = 10000
NP = 10240
E = 320000
D_IN = 128
D_H = 128
D_OUT = 64
DS = 64

NC = 2
NS = 16
NW = NC * NS
CH = 128
K = 80
E_PAD = NW * K * CH

ROWS_PT = NP // NS
SUB = ROWS_PT // CH

BM = 1280
GM = NP // BM


@functools.lru_cache(maxsize=None)
def _make_segsum(ntab, with_deg):
  mesh = plsc.VectorSubcoreMesh(core_axis_name="c", subcore_axis_name="s",
                                num_cores=NC, num_subcores=NS)
  out_type = [jax.ShapeDtypeStruct((NC, NP, DS), jnp.float32)
              for _ in range(ntab)]
  if with_deg:
    out_type.append(jax.ShapeDtypeStruct((NC, NP), jnp.float32))
  scratch = [
      pltpu.VMEM((K + 1, CH), jnp.int32),
      pltpu.VMEM((K, CH), jnp.int32),
      pltpu.VMEM((CH, DS), jnp.float32),
      pltpu.VMEM((CH, DS), jnp.float32),
      pltpu.VMEM((CH,), jnp.float32),
      pltpu.VMEM_SHARED((NP, DS), jnp.float32),
  ]
  if with_deg:
    scratch.append(pltpu.VMEM_SHARED((NP,), jnp.float32))
  scratch += [pltpu.SemaphoreType.DMA, pltpu.SemaphoreType.DMA]

  def body(*args):
    tabs = args[:ntab]
    srcp, dstp = args[ntab], args[ntab + 1]
    rest = args[ntab + 2:]
    outs = rest[:ntab]
    rest = rest[ntab:]
    if with_deg:
      deg_out = rest[0]
      idx_s, idx_d, buf_a, buf_b, ones, acc, deg, sem_a, sem_b = rest[1:]
    else:
      idx_s, idx_d, buf_a, buf_b, ones, acc, sem_a, sem_b = rest
      deg = None
      deg_out = None

    c = lax.axis_index("c")
    s = lax.axis_index("s")
    w = c * NS + s
    base = s * ROWS_PT

    zf = jnp.zeros((16,), jnp.float32)

    def zrow(i, _):
      for jj in range(DS // 16):
        buf_a[i, pl.ds(jj * 16, 16)] = zf
      return 0
    lax.fori_loop(0, CH, zrow, 0)

    if with_deg:
      for jj in range(CH // 16):
        ones[pl.ds(jj * 16, 16)] = zf
      for b in range(SUB):
        pltpu.sync_copy(ones, deg.at[pl.ds(base + b * CH, CH)])
      of = jnp.ones((16,), jnp.float32)
      for jj in range(CH // 16):
        ones[pl.ds(jj * 16, 16)] = of

    pltpu.sync_copy(srcp.at[w], idx_s.at[pl.ds(0, K)])
    pltpu.sync_copy(dstp.at[w], idx_d)
    zi = jnp.zeros((16,), jnp.int32)
    for jj in range(CH // 16):
      idx_s[K, pl.ds(jj * 16, 16)] = zi

    for t in range(ntab):
      tbl = tabs[t]
      out = outs[t]
      first = t == 0

      for b in range(SUB):
        pltpu.sync_copy(buf_a, acc.at[pl.ds(base + b * CH, CH)])
      plsc.subcore_barrier()

      pltpu.async_copy(tbl.at[idx_s.at[0]], buf_a, sem_a)

      def step(jj, _):
        j0 = 2 * jj
        j1 = j0 + 1
        j2 = j0 + 2
        pltpu.async_copy(tbl.at[idx_s.at[j1]], buf_b, sem_b)
        pltpu.make_async_copy(tbl.at[idx_s.at[j0]], buf_a, sem_a).wait()
        pltpu.sync_copy(buf_a, acc.at[idx_d.at[j0]], add=True)
        if with_deg and first:
          pltpu.sync_copy(ones, deg.at[idx_d.at[j0]], add=True)
        pltpu.async_copy(tbl.at[idx_s.at[j2]], buf_a, sem_a)
        pltpu.make_async_copy(tbl.at[idx_s.at[j1]], buf_b, sem_b).wait()
        pltpu.sync_copy(buf_b, acc.at[idx_d.at[j1]], add=True)
        if with_deg and first:
          pltpu.sync_copy(ones, deg.at[idx_d.at[j1]], add=True)
        return 0
      lax.fori_loop(0, K // 2, step, 0)

      pltpu.make_async_copy(tbl.at[idx_s.at[K]], buf_a, sem_a).wait()

      plsc.subcore_barrier()

      for b in range(SUB):
        off = base + b * CH
        pltpu.sync_copy(acc.at[pl.ds(off, CH)], buf_b)
        pltpu.sync_copy(buf_b, out.at[c, pl.ds(off, CH)])

    if with_deg:
      for b in range(SUB):
        off = base + b * CH
        pltpu.sync_copy(deg.at[pl.ds(off, CH)], ones)
        pltpu.sync_copy(ones, deg_out.at[c, pl.ds(off, CH)])

  return pl.kernel(body, out_type=tuple(out_type), mesh=mesh,
                   scratch_types=scratch,
                   compiler_params=pltpu.CompilerParams(
                       use_tc_tiling_on_sc=False))


def _tc1_body(x_ref, wla_ref, wlb_ref, wr_ref, b_ref, p1a_ref, p1b_ref,
              pre1_ref):
  xb = x_ref[...]
  p1a_ref[...] = jnp.dot(xb, wla_ref[...], preferred_element_type=jnp.float32)
  p1b_ref[...] = jnp.dot(xb, wlb_ref[...], preferred_element_type=jnp.float32)
  pre1_ref[...] = (jnp.dot(xb, wr_ref[...], preferred_element_type=jnp.float32)
                   + b_ref[...])


_tc1 = pl.pallas_call(
    _tc1_body,
    grid=(GM,),
    in_specs=[
        pl.BlockSpec((BM, D_IN), lambda i: (i, 0)),
        pl.BlockSpec((D_IN, DS), lambda i: (0, 0)),
        pl.BlockSpec((D_IN, DS), lambda i: (0, 0)),
        pl.BlockSpec((D_IN, D_H), lambda i: (0, 0)),
        pl.BlockSpec((1, D_H), lambda i: (0, 0)),
    ],
    out_specs=[
        pl.BlockSpec((BM, DS), lambda i: (i, 0)),
        pl.BlockSpec((BM, DS), lambda i: (i, 0)),
        pl.BlockSpec((BM, D_H), lambda i: (i, 0)),
    ],
    out_shape=[
        jax.ShapeDtypeStruct((NP, DS), jnp.float32),
        jax.ShapeDtypeStruct((NP, DS), jnp.float32),
        jax.ShapeDtypeStruct((NP, D_H), jnp.float32),
    ],
)


def _tc2_body(sa0_ref, sa1_ref, sb0_ref, sb1_ref, dega_ref, degb_ref,
              pre1_ref, wl_ref, wr_ref, b2_ref, p2_ref, pre2_ref):
  deg = jnp.maximum(dega_ref[...] + degb_ref[...], 1.0)
  inv = 1.0 / deg
  aggr_lo = (sa0_ref[...] + sa1_ref[...]) * inv
  aggr_hi = (sb0_ref[...] + sb1_ref[...]) * inv
  pre1 = pre1_ref[...]
  h_lo = jnp.maximum(aggr_lo + pre1[:, :DS], 0.0)
  h_hi = jnp.maximum(aggr_hi + pre1[:, DS:], 0.0)
  h = jnp.concatenate([h_lo, h_hi], axis=1)
  p2_ref[...] = jnp.dot(h, wl_ref[...], preferred_element_type=jnp.float32)
  pre2_ref[...] = (jnp.dot(h, wr_ref[...], preferred_element_type=jnp.float32)
                   + b2_ref[...])


_tc2 = pl.pallas_call(
    _tc2_body,
    grid=(GM,),
    in_specs=[
        pl.BlockSpec((BM, DS), lambda i: (i, 0)),
        pl.BlockSpec((BM, DS), lambda i: (i, 0)),
        pl.BlockSpec((BM, DS), lambda i: (i, 0)),
        pl.BlockSpec((BM, DS), lambda i: (i, 0)),
        pl.BlockSpec((BM, 1), lambda i: (i, 0)),
        pl.BlockSpec((BM, 1), lambda i: (i, 0)),
        pl.BlockSpec((BM, D_H), lambda i: (i, 0)),
        pl.BlockSpec((D_H, D_OUT), lambda i: (0, 0)),
        pl.BlockSpec((D_H, D_OUT), lambda i: (0, 0)),
        pl.BlockSpec((1, D_OUT), lambda i: (0, 0)),
    ],
    out_specs=[
        pl.BlockSpec((BM, D_OUT), lambda i: (i, 0)),
        pl.BlockSpec((BM, D_OUT), lambda i: (i, 0)),
    ],
    out_shape=[jax.ShapeDtypeStruct((NP, D_OUT), jnp.float32)] * 2,
)


def _tc3_body(s2a_ref, s2b_ref, dega_ref, degb_ref, pre2_ref, y_ref, m_ref,
              out_ref, accs):
  i = pl.program_id(0)
  deg = jnp.maximum(dega_ref[...] + degb_ref[...], 1.0)
  z = (s2a_ref[...] + s2b_ref[...]) / deg + pre2_ref[...]
  zmax = jnp.max(z, axis=1, keepdims=True)
  lse = jnp.log(jnp.sum(jnp.exp(z - zmax), axis=1, keepdims=True)) + zmax
  logp = z - lse
  onehot = lax.broadcasted_iota(jnp.int32, z.shape, 1) == y_ref[...]
  picked = jnp.sum(jnp.where(onehot, logp, 0.0), axis=1)
  mv = m_ref[...][:, 0]
  pn = jnp.sum(picked * mv)
  pm = jnp.sum(mv)

  @pl.when(i == 0)
  def _():
    accs[0] = pn
    accs[1] = pm

  @pl.when(i > 0)
  def _():
    accs[0] += pn
    accs[1] += pm

  @pl.when(i == GM - 1)
  def _():
    out_ref[...] = jnp.full((1, 1), -accs[0] / jnp.maximum(accs[1], 1.0),
                            jnp.float32)


_tc3 = pl.pallas_call(
    _tc3_body,
    grid=(GM,),
    in_specs=[
        pl.BlockSpec((BM, D_OUT), lambda i: (i, 0)),
        pl.BlockSpec((BM, D_OUT), lambda i: (i, 0)),
        pl.BlockSpec((BM, 1), lambda i: (i, 0)),
        pl.BlockSpec((BM, 1), lambda i: (i, 0)),
        pl.BlockSpec((BM, D_OUT), lambda i: (i, 0)),
        pl.BlockSpec((BM, 1), lambda i: (i, 0)),
        pl.BlockSpec((BM, 1), lambda i: (i, 0)),
    ],
    out_specs=pl.BlockSpec((1, 1), lambda i: (0, 0)),
    out_shape=jax.ShapeDtypeStruct((1, 1), jnp.float32),
    scratch_shapes=[pltpu.SMEM((2,), jnp.float32)],
)


def kernel(x, edge_index, y, train_mask, Wl1, bl1, Wr1, br1, Wl2, bl2, Wr2,
           br2):
  src = edge_index[0]
  dst = edge_index[1]
  pad_e = E_PAD - E
  srcp = jnp.concatenate([src, jnp.zeros((pad_e,), jnp.int32)])
  srcp = srcp.reshape(NW, K, CH)
  dstp = jnp.concatenate([dst, jnp.full((pad_e,), N, jnp.int32)])
  dstp = dstp.reshape(NW, K, CH)

  xp = jnp.pad(x, ((0, NP - N), (0, 0)))
  b1 = (bl1 + br1).reshape(1, D_H)
  b2 = (bl2 + br2).reshape(1, D_OUT)

  p1a, p1b, pre1 = _tc1(xp, Wl1[:, :DS], Wl1[:, DS:], Wr1, b1)
  sa, sb, degp = _make_segsum(2, True)(p1a, p1b, srcp, dstp)
  dega = degp[0].reshape(NP, 1)
  degb = degp[1].reshape(NP, 1)
  p2, pre2 = _tc2(sa[0], sa[1], sb[0], sb[1], dega, degb, pre1, Wl2, Wr2, b2)
  (parts2,) = _make_segsum(1, False)(p2, srcp, dstp)

  yp = jnp.pad(y, (0, NP - N)).reshape(NP, 1)
  mp = jnp.pad(train_mask.astype(jnp.float32), (0, NP - N)).reshape(NP, 1)
  loss = _tc3(parts2[0], parts2[1], dega, degb, pre2, yp, mp)
  return loss.reshape(1)
